# baseline (speedup 1.0000x reference)
import functools

import jax
import jax.numpy as jnp
from jax import lax
from jax.experimental import pallas as pl
from jax.experimental.pallas import tpu as pltpu
from jax.experimental.pallas import tpu_sc as plsc

E = 8
TOPK = 2
D = 1024
N = 2048
BM = 256
NB = N // BM
BR = 128
NPAIR = N * TOPK
NBLK = NPAIR // BR + E
PADROWS = NBLK * BR
NW = 32
TPW = N // NW
CHW = 32
NEG = -1e30


def _a_body(x_ref, W1_ref, b1_ref, Wg_ref, bg_ref,
            h_ref, w1_ref, w2_ref, e1_ref, e2_ref, r1_ref, r2_ref,
            counts_ref, gstd_ref, run_ref):
    i = pl.program_id(0)
    x = x_ref[...]
    h = jnp.maximum(
        jnp.dot(x, W1_ref[...], preferred_element_type=jnp.float32)
        + b1_ref[...], 0.0)
    h_ref[...] = h
    logits = (jnp.dot(h, Wg_ref[...], preferred_element_type=jnp.float32)
              + bg_ref[...])
    ii = lax.broadcasted_iota(jnp.int32, (BM, E), 1)
    m1 = jnp.max(logits, axis=1, keepdims=True)
    i1 = jnp.min(jnp.where(logits == m1, ii, E), axis=1, keepdims=True)
    lm = jnp.where(ii == i1, NEG, logits)
    m2 = jnp.max(lm, axis=1, keepdims=True)
    i2 = jnp.min(jnp.where(lm == m2, ii, E), axis=1, keepdims=True)
    w1 = 1.0 / (1.0 + jnp.exp(m2 - m1))

    g = jnp.exp(logits - m1)
    g = g / jnp.sum(g, axis=1, keepdims=True)
    mu = jnp.mean(g, axis=1, keepdims=True)
    var = jnp.sum((g - mu) ** 2, axis=1, keepdims=True) / (E - 1)
    part = jnp.sum(jnp.sqrt(var)) / N

    @pl.when(i == 0)
    def _():
        gstd_ref[...] = jnp.zeros_like(gstd_ref)
        run_ref[...] = jnp.zeros_like(run_ref)

    gstd_ref[...] += jnp.reshape(part, (1, 1))

    run = run_ref[...]
    O1 = (ii == i1).astype(jnp.float32)
    O2 = (ii == i2).astype(jnp.float32)
    ri = lax.broadcasted_iota(jnp.int32, (BM, BM), 0)
    ci = lax.broadcasted_iota(jnp.int32, (BM, BM), 1)
    T = (ri > ci).astype(jnp.float32)
    cb1 = jnp.dot(T, O1, preferred_element_type=jnp.float32)
    s1 = jnp.sum(O1, axis=0, keepdims=True)
    cb2 = jnp.dot(T, O2, preferred_element_type=jnp.float32) + s1
    rank1 = jnp.sum(O1 * (cb1 + run), axis=1, keepdims=True)
    rank2 = jnp.sum(O2 * (cb2 + run), axis=1, keepdims=True)
    run = run + s1 + jnp.sum(O2, axis=0, keepdims=True)
    run_ref[...] = run
    counts_ref[...] = run.astype(jnp.int32)

    w1_ref[...] = w1
    w2_ref[...] = 1.0 - w1
    e1_ref[...] = i1
    e2_ref[...] = i2
    r1_ref[...] = rank1.astype(jnp.int32)
    r2_ref[...] = rank2.astype(jnp.int32)


def _a2_body(counts_ref, nact_ref, bmap_ref, blkoff_ref):
    c = counts_ref[...]
    nblk = jnp.right_shift(c + (BR - 1), 7)
    nblk_f = nblk.astype(jnp.float32)
    ei = lax.broadcasted_iota(jnp.int32, (E, E), 0)
    ej = lax.broadcasted_iota(jnp.int32, (E, E), 1)
    T8 = (ei < ej).astype(jnp.float32)
    blkoff = jnp.dot(nblk_f, T8,
                     preferred_element_type=jnp.float32)
    blkoff_i = blkoff.astype(jnp.int32)
    nact = jnp.sum(nblk, axis=1, keepdims=True)
    nact_ref[...] = nact
    blkoff_ref[...] = jnp.concatenate(
        [blkoff_i * BR, jnp.zeros((1, 16 - E), jnp.int32)], axis=1)
    b_io = lax.broadcasted_iota(jnp.int32, (1, 64), 1)
    bcl = jnp.minimum(b_io, nact - 1)
    acc = jnp.zeros((1, 64), jnp.int32)
    for e in range(E):
        acc = acc + (bcl >= blkoff_i[0, e]).astype(jnp.int32)
    bmap_ref[...] = acc - 1


def _b_body(e1_hbm, e2_hbm, r1_hbm, r2_hbm, w1_hbm, w2_hbm, blkoff_hbm,
            h_hbm, d1_out, d2_out, ws_out, hs_out,
            blk_v, ei_v, ri_v, wv_v, di1_v, di2_v, rows_v):
    cidx = lax.axis_index("c")
    sidx = lax.axis_index("s")
    wid = sidx * 2 + cidx
    base = wid * TPW
    pltpu.sync_copy(blkoff_hbm, blk_v)
    pltpu.sync_copy(e1_hbm.at[pl.ds(base, TPW)], ei_v)
    pltpu.sync_copy(r1_hbm.at[pl.ds(base, TPW)], ri_v)
    for k in range(TPW // 16):
        sl = pl.ds(16 * k, 16)
        off = plsc.load_gather(blk_v, [ei_v[sl]])
        di1_v[sl] = off + ri_v[sl]
    pltpu.sync_copy(e2_hbm.at[pl.ds(base, TPW)], ei_v)
    pltpu.sync_copy(r2_hbm.at[pl.ds(base, TPW)], ri_v)
    for k in range(TPW // 16):
        sl = pl.ds(16 * k, 16)
        off = plsc.load_gather(blk_v, [ei_v[sl]])
        di2_v[sl] = off + ri_v[sl]
    pltpu.sync_copy(di1_v, d1_out.at[pl.ds(base, TPW)])
    pltpu.sync_copy(di2_v, d2_out.at[pl.ds(base, TPW)])
    pltpu.sync_copy(w1_hbm.at[pl.ds(base, TPW)], wv_v)
    pltpu.sync_copy(wv_v, ws_out.at[di1_v])
    pltpu.sync_copy(w2_hbm.at[pl.ds(base, TPW)], wv_v)
    pltpu.sync_copy(wv_v, ws_out.at[di2_v])
    pltpu.sync_copy(h_hbm.at[pl.ds(base, TPW)], rows_v)
    pltpu.sync_copy(rows_v, hs_out.at[di1_v])
    pltpu.sync_copy(rows_v, hs_out.at[di2_v])


def _c_body(bmap_sref, nact_sref, hs_ref, We_ref, be_ref, ws_ref, ys_ref):
    b = pl.program_id(0)

    @pl.when(b < nact_sref[0])
    def _():
        y = (jnp.dot(hs_ref[...], We_ref[0],
                     preferred_element_type=jnp.float32) + be_ref[0])
        ys_ref[...] = y * ws_ref[...]


def _d_body(ys_hbm, d1_hbm, d2_hbm, moe_hbm, d_v, rows1_v, rows2_v):
    cidx = lax.axis_index("c")
    sidx = lax.axis_index("s")
    wid = sidx * 2 + cidx
    for half in range(TPW // CHW):
        base = wid * TPW + half * CHW
        pltpu.sync_copy(d1_hbm.at[pl.ds(base, CHW)], d_v)
        pltpu.sync_copy(ys_hbm.at[d_v], rows1_v)
        pltpu.sync_copy(d2_hbm.at[pl.ds(base, CHW)], d_v)
        pltpu.sync_copy(ys_hbm.at[d_v], rows2_v)

        def add_col(j, _):
            sl = pl.ds(j * 16, 16)
            for t in range(CHW):
                rows1_v[t, sl] += rows2_v[t, sl]
            return 0

        lax.fori_loop(0, D // 16, add_col, 0)
        pltpu.sync_copy(rows1_v, moe_hbm.at[pl.ds(base, CHW)])


def _e_body(moe_ref, W2_ref, b2_ref, out_ref):
    moe = jnp.maximum(moe_ref[...], 0.0)
    out_ref[...] = (jnp.dot(moe, W2_ref[...],
                            preferred_element_type=jnp.float32)
                    + b2_ref[...])


def _sc_mesh():
    return plsc.VectorSubcoreMesh(core_axis_name="c", subcore_axis_name="s",
                                  num_cores=2, num_subcores=16)


def _dispatch(e1, e2, r1, r2, w1, w2, blkoff, h):
    f32 = jnp.float32
    i32 = jnp.int32
    return pl.kernel(
        _b_body,
        out_type=[
            jax.ShapeDtypeStruct((N,), i32),
            jax.ShapeDtypeStruct((N,), i32),
            jax.ShapeDtypeStruct((PADROWS,), f32),
            jax.ShapeDtypeStruct((PADROWS, D), f32),
        ],
        mesh=_sc_mesh(),
        compiler_params=pltpu.CompilerParams(needs_layout_passes=False),
        scratch_types=[
            pltpu.VMEM((16,), i32),
            pltpu.VMEM((TPW,), i32),
            pltpu.VMEM((TPW,), i32),
            pltpu.VMEM((TPW,), f32),
            pltpu.VMEM((TPW,), i32),
            pltpu.VMEM((TPW,), i32),
            pltpu.VMEM((TPW, D), f32),
        ],
    )(e1, e2, r1, r2, w1, w2, blkoff, h)


def _combine(ys, d1, d2):
    return pl.kernel(
        _d_body,
        out_type=jax.ShapeDtypeStruct((N, D), jnp.float32),
        mesh=_sc_mesh(),
        scratch_types=[
            pltpu.VMEM((CHW,), jnp.int32),
            pltpu.VMEM((CHW, D), jnp.float32),
            pltpu.VMEM((CHW, D), jnp.float32),
        ],
    )(ys, d1, d2)


def kernel(x, W1, b1, Wg, bg, We, be, W2, b2):
    f32 = jnp.float32
    i32 = jnp.int32

    h, w1, w2, e1, e2, r1, r2, counts, gstd = pl.pallas_call(
        _a_body,
        grid=(NB,),
        in_specs=[
            pl.BlockSpec((BM, D), lambda i: (i, 0)),
            pl.BlockSpec((D, D), lambda i: (0, 0)),
            pl.BlockSpec((1, D), lambda i: (0, 0)),
            pl.BlockSpec((D, E), lambda i: (0, 0)),
            pl.BlockSpec((1, E), lambda i: (0, 0)),
        ],
        out_specs=[
            pl.BlockSpec((BM, D), lambda i: (i, 0)),
            pl.BlockSpec((BM, 1), lambda i: (i, 0)),
            pl.BlockSpec((BM, 1), lambda i: (i, 0)),
            pl.BlockSpec((BM, 1), lambda i: (i, 0)),
            pl.BlockSpec((BM, 1), lambda i: (i, 0)),
            pl.BlockSpec((BM, 1), lambda i: (i, 0)),
            pl.BlockSpec((BM, 1), lambda i: (i, 0)),
            pl.BlockSpec((1, E), lambda i: (0, 0)),
            pl.BlockSpec((1, 1), lambda i: (0, 0)),
        ],
        out_shape=[
            jax.ShapeDtypeStruct((N, D), f32),
            jax.ShapeDtypeStruct((N, 1), f32),
            jax.ShapeDtypeStruct((N, 1), f32),
            jax.ShapeDtypeStruct((N, 1), i32),
            jax.ShapeDtypeStruct((N, 1), i32),
            jax.ShapeDtypeStruct((N, 1), i32),
            jax.ShapeDtypeStruct((N, 1), i32),
            jax.ShapeDtypeStruct((1, E), i32),
            jax.ShapeDtypeStruct((1, 1), f32),
        ],
        scratch_shapes=[pltpu.VMEM((1, E), f32)],
    )(x, W1, b1.reshape(1, D), Wg, bg.reshape(1, E))

    nact, bmap, blkoff = pl.pallas_call(
        _a2_body,
        grid=(1,),
        in_specs=[pl.BlockSpec((1, E), lambda i: (0, 0))],
        out_specs=[
            pl.BlockSpec((1, 1), lambda i: (0, 0)),
            pl.BlockSpec((1, 64), lambda i: (0, 0)),
            pl.BlockSpec((1, 16), lambda i: (0, 0)),
        ],
        out_shape=[
            jax.ShapeDtypeStruct((1, 1), i32),
            jax.ShapeDtypeStruct((1, 64), i32),
            jax.ShapeDtypeStruct((1, 16), i32),
        ],
    )(counts)

    d1, d2, ws, hs = _dispatch(
        e1.reshape(N), e2.reshape(N), r1.reshape(N), r2.reshape(N),
        w1.reshape(N), w2.reshape(N), blkoff.reshape(16), h)

    grid_spec = pltpu.PrefetchScalarGridSpec(
        num_scalar_prefetch=2,
        grid=(NBLK,),
        in_specs=[
            pl.BlockSpec((BR, D),
                         lambda b, bmap, nact: (jnp.minimum(b, nact[0] - 1), 0)),
            pl.BlockSpec((1, D, D),
                         lambda b, bmap, nact: (bmap[b], 0, 0)),
            pl.BlockSpec((1, 1, D),
                         lambda b, bmap, nact: (bmap[b], 0, 0)),
            pl.BlockSpec((BR, 1),
                         lambda b, bmap, nact: (jnp.minimum(b, nact[0] - 1), 0)),
        ],
        out_specs=pl.BlockSpec(
            (BR, D), lambda b, bmap, nact: (jnp.minimum(b, nact[0] - 1), 0)),
    )
    ys = pl.pallas_call(
        _c_body,
        grid_spec=grid_spec,
        out_shape=jax.ShapeDtypeStruct((PADROWS, D), f32),
    )(bmap.reshape(64), nact.reshape(1), hs, We, be.reshape(E, 1, D),
      ws.reshape(PADROWS, 1))

    moe = _combine(ys, d1, d2)

    out = pl.pallas_call(
        _e_body,
        grid=(NB,),
        in_specs=[
            pl.BlockSpec((BM, D), lambda i: (i, 0)),
            pl.BlockSpec((D, D), lambda i: (0, 0)),
            pl.BlockSpec((1, D), lambda i: (0, 0)),
        ],
        out_specs=pl.BlockSpec((BM, D), lambda i: (i, 0)),
        out_shape=jax.ShapeDtypeStruct((N, D), f32),
    )(moe, W2, b2.reshape(1, D))

    return out, gstd[0, 0]


# speedup vs baseline: 1.0030x; 1.0030x over previous
import functools

import jax
import jax.numpy as jnp
from jax import lax
from jax.experimental import pallas as pl
from jax.experimental.pallas import tpu as pltpu
from jax.experimental.pallas import tpu_sc as plsc

E = 8
TOPK = 2
D = 1024
N = 2048
BM = 256
NB = N // BM
BR = 128
NPAIR = N * TOPK
NBLK = NPAIR // BR + E
PADROWS = NBLK * BR
NW = 32
TPW = N // NW
CHW = 32
NEG = -1e30


def _a_body(x_ref, W1_ref, b1_ref, Wg_ref, bg_ref,
            h_ref, w1_ref, w2_ref, e1_ref, e2_ref, r1_ref, r2_ref,
            counts_ref, gstd_ref, run_ref):
    i = pl.program_id(0)
    x = x_ref[...]
    h = jnp.maximum(
        jnp.dot(x, W1_ref[...], preferred_element_type=jnp.float32)
        + b1_ref[...], 0.0)
    h_ref[...] = h
    logits = (jnp.dot(h, Wg_ref[...], preferred_element_type=jnp.float32)
              + bg_ref[...])
    ii = lax.broadcasted_iota(jnp.int32, (BM, E), 1)
    m1 = jnp.max(logits, axis=1, keepdims=True)
    i1 = jnp.min(jnp.where(logits == m1, ii, E), axis=1, keepdims=True)
    lm = jnp.where(ii == i1, NEG, logits)
    m2 = jnp.max(lm, axis=1, keepdims=True)
    i2 = jnp.min(jnp.where(lm == m2, ii, E), axis=1, keepdims=True)
    w1 = 1.0 / (1.0 + jnp.exp(m2 - m1))

    g = jnp.exp(logits - m1)
    g = g / jnp.sum(g, axis=1, keepdims=True)
    mu = jnp.mean(g, axis=1, keepdims=True)
    var = jnp.sum((g - mu) ** 2, axis=1, keepdims=True) / (E - 1)
    part = jnp.sum(jnp.sqrt(var)) / N

    @pl.when(i == 0)
    def _():
        gstd_ref[...] = jnp.zeros_like(gstd_ref)
        run_ref[...] = jnp.zeros_like(run_ref)

    gstd_ref[...] += jnp.reshape(part, (1, 1))

    run = run_ref[...]
    O1 = (ii == i1).astype(jnp.float32)
    O2 = (ii == i2).astype(jnp.float32)
    ri = lax.broadcasted_iota(jnp.int32, (BM, BM), 0)
    ci = lax.broadcasted_iota(jnp.int32, (BM, BM), 1)
    T = (ri > ci).astype(jnp.float32)
    cb1 = jnp.dot(T, O1, preferred_element_type=jnp.float32)
    s1 = jnp.sum(O1, axis=0, keepdims=True)
    cb2 = jnp.dot(T, O2, preferred_element_type=jnp.float32) + s1
    rank1 = jnp.sum(O1 * (cb1 + run), axis=1, keepdims=True)
    rank2 = jnp.sum(O2 * (cb2 + run), axis=1, keepdims=True)
    run = run + s1 + jnp.sum(O2, axis=0, keepdims=True)
    run_ref[...] = run
    counts_ref[...] = run.astype(jnp.int32)

    w1_ref[...] = w1
    w2_ref[...] = 1.0 - w1
    e1_ref[...] = i1
    e2_ref[...] = i2
    r1_ref[...] = rank1.astype(jnp.int32)
    r2_ref[...] = rank2.astype(jnp.int32)


def _a2_body(counts_ref, nact_ref, bmap_ref, blkoff_ref):
    c = counts_ref[...]
    nblk = jnp.right_shift(c + (BR - 1), 7)
    nblk_f = nblk.astype(jnp.float32)
    ei = lax.broadcasted_iota(jnp.int32, (E, E), 0)
    ej = lax.broadcasted_iota(jnp.int32, (E, E), 1)
    T8 = (ei < ej).astype(jnp.float32)
    blkoff = jnp.dot(nblk_f, T8,
                     preferred_element_type=jnp.float32)
    blkoff_i = blkoff.astype(jnp.int32)
    nact = jnp.sum(nblk, axis=1, keepdims=True)
    nact_ref[...] = nact
    blkoff_ref[...] = jnp.concatenate(
        [blkoff_i * BR, jnp.zeros((1, 16 - E), jnp.int32)], axis=1)
    b_io = lax.broadcasted_iota(jnp.int32, (1, 64), 1)
    bcl = jnp.minimum(b_io, nact - 1)
    acc = jnp.zeros((1, 64), jnp.int32)
    for e in range(E):
        acc = acc + (bcl >= blkoff_i[0, e]).astype(jnp.int32)
    bmap_ref[...] = acc - 1


def _b_body(e1_hbm, e2_hbm, r1_hbm, r2_hbm, w1_hbm, w2_hbm, blkoff_hbm,
            h_hbm, d1_out, d2_out, ws_out, hs_out,
            blk_v, ei_v, ri_v, wv_v, di1_v, di2_v, rows_v):
    cidx = lax.axis_index("c")
    sidx = lax.axis_index("s")
    wid = sidx * 2 + cidx
    base = wid * TPW
    pltpu.sync_copy(blkoff_hbm, blk_v)
    pltpu.sync_copy(e1_hbm.at[pl.ds(base, TPW)], ei_v)
    pltpu.sync_copy(r1_hbm.at[pl.ds(base, TPW)], ri_v)
    for k in range(TPW // 16):
        sl = pl.ds(16 * k, 16)
        off = plsc.load_gather(blk_v, [ei_v[sl]])
        di1_v[sl] = off + ri_v[sl]
    pltpu.sync_copy(e2_hbm.at[pl.ds(base, TPW)], ei_v)
    pltpu.sync_copy(r2_hbm.at[pl.ds(base, TPW)], ri_v)
    for k in range(TPW // 16):
        sl = pl.ds(16 * k, 16)
        off = plsc.load_gather(blk_v, [ei_v[sl]])
        di2_v[sl] = off + ri_v[sl]
    pltpu.sync_copy(di1_v, d1_out.at[pl.ds(base, TPW)])
    pltpu.sync_copy(di2_v, d2_out.at[pl.ds(base, TPW)])
    pltpu.sync_copy(w1_hbm.at[pl.ds(base, TPW)], wv_v)
    pltpu.sync_copy(wv_v, ws_out.at[di1_v])
    pltpu.sync_copy(w2_hbm.at[pl.ds(base, TPW)], wv_v)
    pltpu.sync_copy(wv_v, ws_out.at[di2_v])
    pltpu.sync_copy(h_hbm.at[pl.ds(base, TPW)], rows_v)
    pltpu.sync_copy(rows_v, hs_out.at[di1_v])
    pltpu.sync_copy(rows_v, hs_out.at[di2_v])


def _c_body(bmap_sref, nact_sref, hs_ref, We_ref, be_ref, ws_ref, ys_ref):
    b = pl.program_id(0)

    @pl.when(b < nact_sref[0])
    def _():
        e = bmap_sref[b]
        y = (jnp.dot(hs_ref[...], We_ref[e],
                     preferred_element_type=jnp.float32) + be_ref[e])
        ys_ref[...] = y * ws_ref[...]


def _d_body(ys_hbm, d1_hbm, d2_hbm, moe_hbm, d_v, rows1_v, rows2_v):
    cidx = lax.axis_index("c")
    sidx = lax.axis_index("s")
    wid = sidx * 2 + cidx
    for half in range(TPW // CHW):
        base = wid * TPW + half * CHW
        pltpu.sync_copy(d1_hbm.at[pl.ds(base, CHW)], d_v)
        pltpu.sync_copy(ys_hbm.at[d_v], rows1_v)
        pltpu.sync_copy(d2_hbm.at[pl.ds(base, CHW)], d_v)
        pltpu.sync_copy(ys_hbm.at[d_v], rows2_v)

        def add_col(j, _):
            sl = pl.ds(j * 16, 16)
            for t in range(CHW):
                rows1_v[t, sl] += rows2_v[t, sl]
            return 0

        lax.fori_loop(0, D // 16, add_col, 0)
        pltpu.sync_copy(rows1_v, moe_hbm.at[pl.ds(base, CHW)])


def _e_body(moe_ref, W2_ref, b2_ref, out_ref):
    moe = jnp.maximum(moe_ref[...], 0.0)
    out_ref[...] = (jnp.dot(moe, W2_ref[...],
                            preferred_element_type=jnp.float32)
                    + b2_ref[...])


def _sc_mesh():
    return plsc.VectorSubcoreMesh(core_axis_name="c", subcore_axis_name="s",
                                  num_cores=2, num_subcores=16)


def _dispatch(e1, e2, r1, r2, w1, w2, blkoff, h):
    f32 = jnp.float32
    i32 = jnp.int32
    return pl.kernel(
        _b_body,
        out_type=[
            jax.ShapeDtypeStruct((N,), i32),
            jax.ShapeDtypeStruct((N,), i32),
            jax.ShapeDtypeStruct((PADROWS,), f32),
            jax.ShapeDtypeStruct((PADROWS, D), f32),
        ],
        mesh=_sc_mesh(),
        compiler_params=pltpu.CompilerParams(needs_layout_passes=False),
        scratch_types=[
            pltpu.VMEM((16,), i32),
            pltpu.VMEM((TPW,), i32),
            pltpu.VMEM((TPW,), i32),
            pltpu.VMEM((TPW,), f32),
            pltpu.VMEM((TPW,), i32),
            pltpu.VMEM((TPW,), i32),
            pltpu.VMEM((TPW, D), f32),
        ],
    )(e1, e2, r1, r2, w1, w2, blkoff, h)


def _combine(ys, d1, d2):
    return pl.kernel(
        _d_body,
        out_type=jax.ShapeDtypeStruct((N, D), jnp.float32),
        mesh=_sc_mesh(),
        scratch_types=[
            pltpu.VMEM((CHW,), jnp.int32),
            pltpu.VMEM((CHW, D), jnp.float32),
            pltpu.VMEM((CHW, D), jnp.float32),
        ],
    )(ys, d1, d2)


def kernel(x, W1, b1, Wg, bg, We, be, W2, b2):
    f32 = jnp.float32
    i32 = jnp.int32

    h, w1, w2, e1, e2, r1, r2, counts, gstd = pl.pallas_call(
        _a_body,
        grid=(NB,),
        in_specs=[
            pl.BlockSpec((BM, D), lambda i: (i, 0)),
            pl.BlockSpec((D, D), lambda i: (0, 0)),
            pl.BlockSpec((1, D), lambda i: (0, 0)),
            pl.BlockSpec((D, E), lambda i: (0, 0)),
            pl.BlockSpec((1, E), lambda i: (0, 0)),
        ],
        out_specs=[
            pl.BlockSpec((BM, D), lambda i: (i, 0)),
            pl.BlockSpec((BM, 1), lambda i: (i, 0)),
            pl.BlockSpec((BM, 1), lambda i: (i, 0)),
            pl.BlockSpec((BM, 1), lambda i: (i, 0)),
            pl.BlockSpec((BM, 1), lambda i: (i, 0)),
            pl.BlockSpec((BM, 1), lambda i: (i, 0)),
            pl.BlockSpec((BM, 1), lambda i: (i, 0)),
            pl.BlockSpec((1, E), lambda i: (0, 0)),
            pl.BlockSpec((1, 1), lambda i: (0, 0)),
        ],
        out_shape=[
            jax.ShapeDtypeStruct((N, D), f32),
            jax.ShapeDtypeStruct((N, 1), f32),
            jax.ShapeDtypeStruct((N, 1), f32),
            jax.ShapeDtypeStruct((N, 1), i32),
            jax.ShapeDtypeStruct((N, 1), i32),
            jax.ShapeDtypeStruct((N, 1), i32),
            jax.ShapeDtypeStruct((N, 1), i32),
            jax.ShapeDtypeStruct((1, E), i32),
            jax.ShapeDtypeStruct((1, 1), f32),
        ],
        scratch_shapes=[pltpu.VMEM((1, E), f32)],
    )(x, W1, b1.reshape(1, D), Wg, bg.reshape(1, E))

    nact, bmap, blkoff = pl.pallas_call(
        _a2_body,
        grid=(1,),
        in_specs=[pl.BlockSpec((1, E), lambda i: (0, 0))],
        out_specs=[
            pl.BlockSpec((1, 1), lambda i: (0, 0)),
            pl.BlockSpec((1, 64), lambda i: (0, 0)),
            pl.BlockSpec((1, 16), lambda i: (0, 0)),
        ],
        out_shape=[
            jax.ShapeDtypeStruct((1, 1), i32),
            jax.ShapeDtypeStruct((1, 64), i32),
            jax.ShapeDtypeStruct((1, 16), i32),
        ],
    )(counts)

    d1, d2, ws, hs = _dispatch(
        e1.reshape(N), e2.reshape(N), r1.reshape(N), r2.reshape(N),
        w1.reshape(N), w2.reshape(N), blkoff.reshape(16), h)

    grid_spec = pltpu.PrefetchScalarGridSpec(
        num_scalar_prefetch=2,
        grid=(NBLK,),
        in_specs=[
            pl.BlockSpec((BR, D),
                         lambda b, bmap, nact: (jnp.minimum(b, nact[0] - 1), 0)),
            pl.BlockSpec((E, D, D),
                         lambda b, bmap, nact: (0, 0, 0)),
            pl.BlockSpec((E, 1, D),
                         lambda b, bmap, nact: (0, 0, 0)),
            pl.BlockSpec((BR, 1),
                         lambda b, bmap, nact: (jnp.minimum(b, nact[0] - 1), 0)),
        ],
        out_specs=pl.BlockSpec(
            (BR, D), lambda b, bmap, nact: (jnp.minimum(b, nact[0] - 1), 0)),
    )
    ys = pl.pallas_call(
        _c_body,
        grid_spec=grid_spec,
        out_shape=jax.ShapeDtypeStruct((PADROWS, D), f32),
    )(bmap.reshape(64), nact.reshape(1), hs, We, be.reshape(E, 1, D),
      ws.reshape(PADROWS, 1))

    moe = _combine(ys, d1, d2)

    out = pl.pallas_call(
        _e_body,
        grid=(NB,),
        in_specs=[
            pl.BlockSpec((BM, D), lambda i: (i, 0)),
            pl.BlockSpec((D, D), lambda i: (0, 0)),
            pl.BlockSpec((1, D), lambda i: (0, 0)),
        ],
        out_specs=pl.BlockSpec((BM, D), lambda i: (i, 0)),
        out_shape=jax.ShapeDtypeStruct((N, D), f32),
    )(moe, W2, b2.reshape(1, D))

    return out, gstd[0, 0]


# speedup vs baseline: 2.4347x; 2.4274x over previous
import jax
import jax.numpy as jnp
from jax import lax
from jax.experimental import pallas as pl
from jax.experimental.pallas import tpu as pltpu

E = 8
TOPK = 2
D = 1024
N = 2048
BM = 512
NB = N // BM
NEG = -1e30


def _dense_body(x_ref, W1_ref, b1_ref, Wg_ref, bg_ref, We_ref, be_ref,
                W2_ref, b2_ref, out_ref, gstd_ref):
    i = pl.program_id(0)
    x = x_ref[...]
    h = jnp.maximum(
        jnp.dot(x, W1_ref[...], preferred_element_type=jnp.float32)
        + b1_ref[...], 0.0)
    logits = (jnp.dot(h, Wg_ref[...], preferred_element_type=jnp.float32)
              + bg_ref[...])
    ii = lax.broadcasted_iota(jnp.int32, (BM, E), 1)
    m1 = jnp.max(logits, axis=1, keepdims=True)
    i1 = jnp.min(jnp.where(logits == m1, ii, E), axis=1, keepdims=True)
    lm = jnp.where(ii == i1, NEG, logits)
    m2 = jnp.max(lm, axis=1, keepdims=True)
    i2 = jnp.min(jnp.where(lm == m2, ii, E), axis=1, keepdims=True)
    w1 = 1.0 / (1.0 + jnp.exp(m2 - m1))
    w2 = 1.0 - w1

    g = jnp.exp(logits - m1)
    g = g / jnp.sum(g, axis=1, keepdims=True)
    mu = jnp.mean(g, axis=1, keepdims=True)
    var = jnp.sum((g - mu) ** 2, axis=1, keepdims=True) / (E - 1)
    part = jnp.sum(jnp.sqrt(var)) / N

    @pl.when(i == 0)
    def _():
        gstd_ref[...] = jnp.zeros_like(gstd_ref)

    gstd_ref[...] += jnp.reshape(part, (1, 1))

    be = be_ref[...]
    h_bf = h.astype(jnp.bfloat16)
    acc = jnp.zeros((BM, D), dtype=jnp.float32)
    for e in range(E):
        we = jnp.where(i1 == e, w1, 0.0) + jnp.where(i2 == e, w2, 0.0)
        ye = jnp.dot(h_bf, We_ref[e].astype(jnp.bfloat16),
                     preferred_element_type=jnp.float32)
        acc = acc + we * (ye + be[e][None, :])
    moe = jnp.maximum(acc, 0.0).astype(jnp.bfloat16)
    out = (jnp.dot(moe, W2_ref[...].astype(jnp.bfloat16),
                   preferred_element_type=jnp.float32)
           + b2_ref[...])
    out_ref[...] = out


def kernel(x, W1, b1, Wg, bg, We, be, W2, b2):
    out, gstd = pl.pallas_call(
        _dense_body,
        grid=(NB,),
        in_specs=[
            pl.BlockSpec((BM, D), lambda i: (i, 0)),
            pl.BlockSpec((D, D), lambda i: (0, 0)),
            pl.BlockSpec((1, D), lambda i: (0, 0)),
            pl.BlockSpec((D, E), lambda i: (0, 0)),
            pl.BlockSpec((1, E), lambda i: (0, 0)),
            pl.BlockSpec((E, D, D), lambda i: (0, 0, 0)),
            pl.BlockSpec((E, D), lambda i: (0, 0)),
            pl.BlockSpec((D, D), lambda i: (0, 0)),
            pl.BlockSpec((1, D), lambda i: (0, 0)),
        ],
        out_specs=[
            pl.BlockSpec((BM, D), lambda i: (i, 0)),
            pl.BlockSpec((1, 1), lambda i: (0, 0)),
        ],
        out_shape=[
            jax.ShapeDtypeStruct((N, D), jnp.float32),
            jax.ShapeDtypeStruct((1, 1), jnp.float32),
        ],
    )(x, W1, b1.reshape(1, D), Wg, bg.reshape(1, E), We, be, W2,
      b2.reshape(1, D))
    return out, gstd[0, 0]


# speedup vs baseline: 2.4413x; 1.0027x over previous
import jax
import jax.numpy as jnp
from jax import lax
from jax.experimental import pallas as pl
from jax.experimental.pallas import tpu as pltpu

E = 8
TOPK = 2
D = 1024
N = 2048
BM = 512
NB = N // BM
NEG = -1e30


def _dense_body(x_ref, W1_ref, b1_ref, Wg_ref, bg_ref, We_ref, be_ref,
                W2_ref, b2_ref, out_ref, gstd_ref):
    i = pl.program_id(0)
    x = x_ref[...].astype(jnp.bfloat16)
    h = jnp.maximum(
        jnp.dot(x, W1_ref[...].astype(jnp.bfloat16),
                preferred_element_type=jnp.float32)
        + b1_ref[...], 0.0)
    logits = (jnp.dot(h, Wg_ref[...], preferred_element_type=jnp.float32)
              + bg_ref[...])
    ii = lax.broadcasted_iota(jnp.int32, (BM, E), 1)
    m1 = jnp.max(logits, axis=1, keepdims=True)
    i1 = jnp.min(jnp.where(logits == m1, ii, E), axis=1, keepdims=True)
    lm = jnp.where(ii == i1, NEG, logits)
    m2 = jnp.max(lm, axis=1, keepdims=True)
    i2 = jnp.min(jnp.where(lm == m2, ii, E), axis=1, keepdims=True)
    w1 = 1.0 / (1.0 + jnp.exp(m2 - m1))
    w2 = 1.0 - w1

    g = jnp.exp(logits - m1)
    g = g / jnp.sum(g, axis=1, keepdims=True)
    mu = jnp.mean(g, axis=1, keepdims=True)
    var = jnp.sum((g - mu) ** 2, axis=1, keepdims=True) / (E - 1)
    part = jnp.sum(jnp.sqrt(var)) / N

    @pl.when(i == 0)
    def _():
        gstd_ref[...] = jnp.zeros_like(gstd_ref)

    gstd_ref[...] += jnp.reshape(part, (1, 1))

    be = be_ref[...]
    h_bf = h.astype(jnp.bfloat16)
    acc = jnp.zeros((BM, D), dtype=jnp.float32)
    for e in range(E):
        we = jnp.where(i1 == e, w1, 0.0) + jnp.where(i2 == e, w2, 0.0)
        ye = jnp.dot(h_bf, We_ref[e].astype(jnp.bfloat16),
                     preferred_element_type=jnp.float32)
        acc = acc + we * (ye + be[e][None, :])
    moe = jnp.maximum(acc, 0.0).astype(jnp.bfloat16)
    out = (jnp.dot(moe, W2_ref[...].astype(jnp.bfloat16),
                   preferred_element_type=jnp.float32)
           + b2_ref[...])
    out_ref[...] = out


def kernel(x, W1, b1, Wg, bg, We, be, W2, b2):
    out, gstd = pl.pallas_call(
        _dense_body,
        grid=(NB,),
        in_specs=[
            pl.BlockSpec((BM, D), lambda i: (i, 0)),
            pl.BlockSpec((D, D), lambda i: (0, 0)),
            pl.BlockSpec((1, D), lambda i: (0, 0)),
            pl.BlockSpec((D, E), lambda i: (0, 0)),
            pl.BlockSpec((1, E), lambda i: (0, 0)),
            pl.BlockSpec((E, D, D), lambda i: (0, 0, 0)),
            pl.BlockSpec((E, D), lambda i: (0, 0)),
            pl.BlockSpec((D, D), lambda i: (0, 0)),
            pl.BlockSpec((1, D), lambda i: (0, 0)),
        ],
        out_specs=[
            pl.BlockSpec((BM, D), lambda i: (i, 0)),
            pl.BlockSpec((1, 1), lambda i: (0, 0)),
        ],
        out_shape=[
            jax.ShapeDtypeStruct((N, D), jnp.float32),
            jax.ShapeDtypeStruct((1, 1), jnp.float32),
        ],
    )(x, W1, b1.reshape(1, D), Wg, bg.reshape(1, E), We, be, W2,
      b2.reshape(1, D))
    return out, gstd[0, 0]


# speedup vs baseline: 2.5018x; 1.0248x over previous
import jax
import jax.numpy as jnp
from jax import lax
from jax.experimental import pallas as pl
from jax.experimental.pallas import tpu as pltpu

E = 8
TOPK = 2
D = 1024
N = 2048
BM = 512
NB = N // BM
NEG = -1e30


def _dense_body(x_ref, W1_ref, b1_ref, Wg_ref, bg_ref, We_ref, be_ref,
                W2_ref, b2_ref, out_ref, gstd_ref):
    i = pl.program_id(0)
    x = x_ref[...].astype(jnp.bfloat16)
    h = jnp.maximum(
        jnp.dot(x, W1_ref[...].astype(jnp.bfloat16),
                preferred_element_type=jnp.float32)
        + b1_ref[...], 0.0)
    logits = (jnp.dot(h, Wg_ref[...], preferred_element_type=jnp.float32)
              + bg_ref[...])
    ii = lax.broadcasted_iota(jnp.int32, (BM, E), 1)
    m1 = jnp.max(logits, axis=1, keepdims=True)
    i1 = jnp.min(jnp.where(logits == m1, ii, E), axis=1, keepdims=True)
    lm = jnp.where(ii == i1, NEG, logits)
    m2 = jnp.max(lm, axis=1, keepdims=True)
    i2 = jnp.min(jnp.where(lm == m2, ii, E), axis=1, keepdims=True)
    w1 = 1.0 / (1.0 + jnp.exp(m2 - m1))
    w2 = 1.0 - w1

    be = be_ref[...]
    h_bf = h.astype(jnp.bfloat16)
    acc = jnp.zeros((BM, D), dtype=jnp.float32)
    for e in range(E):
        we = jnp.where(i1 == e, w1, 0.0) + jnp.where(i2 == e, w2, 0.0)
        ye = jnp.dot(h_bf, We_ref[e].astype(jnp.bfloat16),
                     preferred_element_type=jnp.float32)
        acc = acc + we * (ye + be[e][None, :])
    moe = jnp.maximum(acc, 0.0).astype(jnp.bfloat16)

    g = jnp.exp(logits - m1)
    g = g / jnp.sum(g, axis=1, keepdims=True)
    mu = jnp.mean(g, axis=1, keepdims=True)
    var = jnp.sum((g - mu) ** 2, axis=1, keepdims=True) / (E - 1)
    part = jnp.sum(jnp.sqrt(var)) / N

    @pl.when(i == 0)
    def _():
        gstd_ref[...] = jnp.zeros_like(gstd_ref)

    gstd_ref[...] += jnp.reshape(part, (1, 1))
    out = (jnp.dot(moe, W2_ref[...].astype(jnp.bfloat16),
                   preferred_element_type=jnp.float32)
           + b2_ref[...])
    out_ref[...] = out


def kernel(x, W1, b1, Wg, bg, We, be, W2, b2):
    out, gstd = pl.pallas_call(
        _dense_body,
        grid=(NB,),
        in_specs=[
            pl.BlockSpec((BM, D), lambda i: (i, 0)),
            pl.BlockSpec((D, D), lambda i: (0, 0)),
            pl.BlockSpec((1, D), lambda i: (0, 0)),
            pl.BlockSpec((D, E), lambda i: (0, 0)),
            pl.BlockSpec((1, E), lambda i: (0, 0)),
            pl.BlockSpec((E, D, D), lambda i: (0, 0, 0)),
            pl.BlockSpec((E, D), lambda i: (0, 0)),
            pl.BlockSpec((D, D), lambda i: (0, 0)),
            pl.BlockSpec((1, D), lambda i: (0, 0)),
        ],
        out_specs=[
            pl.BlockSpec((BM, D), lambda i: (i, 0)),
            pl.BlockSpec((1, 1), lambda i: (0, 0)),
        ],
        out_shape=[
            jax.ShapeDtypeStruct((N, D), jnp.float32),
            jax.ShapeDtypeStruct((1, 1), jnp.float32),
        ],
    )(x, W1, b1.reshape(1, D), Wg, bg.reshape(1, E), We, be, W2,
      b2.reshape(1, D))
    return out, gstd[0, 0]
